# baseline (speedup 1.0000x reference)
import functools

import jax
import jax.numpy as jnp
from jax import lax
from jax.experimental import pallas as pl
from jax.experimental.pallas import tpu as pltpu
from jax.experimental.pallas import tpu_sc as plsc

NUM_CAT = 26
OUT_DIMS = 16
MOST_CAT = 100000
MIN_SIG = 1e-4

ROWS = NUM_CAT * (MOST_CAT + 1)
TCOLS = ROWS // 128
TEDGE = ROWS - TCOLS * 128
ROWS_PAD = (TCOLS + 1) * 128

NW = 32
CHUNK = 1024
GSIZE = 128
NG = CHUNK // GSIZE

GCOLS = 4
NGRP = TCOLS // GCOLS
GPW = -(-NGRP // NW)

_C = (
    3.5075520537e-06 + MIN_SIG,
    0.99979243573,
    -0.49697791117,
    0.31459053537,
    -0.18878267362,
    0.081726808375,
    -0.017208061121,
)


def _transpose_pair(tv_mu, tv_ps, ov_mu, ov_ps, pb_mu, pb_ps, tb, ob, ncols):
    idxs = [lax.iota(jnp.int32, 16) * 17 + d for d in range(OUT_DIMS)]

    def blk_body(b, carry):
        c0 = tb + b * 16
        bm = pb_mu.at[pl.ds(b * 272, 272)]
        bp = pb_ps.at[pl.ds(b * 272, 272)]
        for d in range(OUT_DIMS):
            plsc.store_scatter(bm, [idxs[d]], tv_mu[d, pl.ds(c0, 16)])
            plsc.store_scatter(bp, [idxs[d]], tv_ps[d, pl.ds(c0, 16)])
        return carry

    lax.fori_loop(0, ncols // 16, blk_body, 0, unroll=2)

    def cmp_body(r, carry):
        p0 = (r // 16) * 272 + (r % 16) * 17
        o0 = ob + r * OUT_DIMS
        ov_mu[pl.ds(o0, OUT_DIMS)] = pb_mu[pl.ds(p0, 16)]
        ov_ps[pl.ds(o0, OUT_DIMS)] = pb_ps[pl.ds(p0, 16)]
        return carry

    lax.fori_loop(0, ncols, cmp_body, 0, unroll=4)


GW = GCOLS * 128
GWORDS = GW * OUT_DIMS


def _detile_body(mu_t, ps_t, mu_tail, ps_tail, mu_lin, ps_lin,
                 tv_mu, tv_ps, ov_mu, ov_ps, pb_mu, pb_ps, sem_in, sem_out):
    wid = lax.axis_index("s") * 2 + lax.axis_index("c")
    g_lo = wid * GPW
    g_hi = jnp.minimum(g_lo + GPW, NGRP)

    def fire_in(g):
        par = (g - g_lo) % 2
        c0 = g * GW
        pltpu.async_copy(mu_t.at[:, pl.ds(c0, GW)],
                         tv_mu.at[:, pl.ds(par * GW, GW)], sem_in)
        pltpu.async_copy(ps_t.at[:, pl.ds(c0, GW)],
                         tv_ps.at[:, pl.ds(par * GW, GW)], sem_in)

    def drain_in(g):
        c0 = g * GW
        pltpu.make_async_copy(mu_t.at[:, pl.ds(c0, GW)],
                              tv_mu.at[:, pl.ds(0, GW)], sem_in).wait()
        pltpu.make_async_copy(ps_t.at[:, pl.ds(c0, GW)],
                              tv_ps.at[:, pl.ds(0, GW)], sem_in).wait()

    def fire_out(g):
        par = (g - g_lo) % 2
        w0 = g * GWORDS
        pltpu.async_copy(ov_mu.at[pl.ds(par * GWORDS, GWORDS)],
                         mu_lin.at[pl.ds(w0, GWORDS)], sem_out)
        pltpu.async_copy(ov_ps.at[pl.ds(par * GWORDS, GWORDS)],
                         ps_lin.at[pl.ds(w0, GWORDS)], sem_out)

    def drain_out(g):
        w0 = g * GWORDS
        pltpu.make_async_copy(ov_mu.at[pl.ds(0, GWORDS)],
                              mu_lin.at[pl.ds(w0, GWORDS)], sem_out).wait()
        pltpu.make_async_copy(ov_ps.at[pl.ds(0, GWORDS)],
                              ps_lin.at[pl.ds(w0, GWORDS)], sem_out).wait()

    fire_in(g_lo)

    def grp_body(g, carry):
        par = (g - g_lo) % 2

        @pl.when(g + 1 < g_hi)
        def _():
            fire_in(g + 1)

        drain_in(g)

        @pl.when(g - 2 >= g_lo)
        def _():
            drain_out(g - 2)

        _transpose_pair(tv_mu, tv_ps, ov_mu, ov_ps, pb_mu, pb_ps,
                        par * GW, par * GWORDS, GW)
        fire_out(g)
        return carry

    lax.fori_loop(g_lo, g_hi, grp_body, 0)
    drain_out(g_hi - 2)
    drain_out(g_hi - 1)

    @pl.when(wid == NW - 1)
    def _edge():
        w0 = TCOLS * 16 * 128
        pltpu.sync_copy(mu_tail, ov_mu.at[pl.ds(0, 2048)])
        pltpu.sync_copy(ov_mu.at[pl.ds(0, 2048)], mu_lin.at[pl.ds(w0, 2048)])
        pltpu.sync_copy(ps_tail, ov_ps.at[pl.ds(0, 2048)])
        pltpu.sync_copy(ov_ps.at[pl.ds(0, 2048)], ps_lin.at[pl.ds(w0, 2048)])


_detile = pl.kernel(
    _detile_body,
    out_type=(
        jax.ShapeDtypeStruct((ROWS_PAD * OUT_DIMS,), jnp.float32),
        jax.ShapeDtypeStruct((ROWS_PAD * OUT_DIMS,), jnp.float32),
    ),
    mesh=plsc.VectorSubcoreMesh(core_axis_name="c", subcore_axis_name="s"),
    compiler_params=pltpu.CompilerParams(use_tc_tiling_on_sc=True,
                                         needs_layout_passes=False),
    scratch_types=[
        pltpu.VMEM((16, 2 * GCOLS * 128), jnp.float32),
        pltpu.VMEM((16, 2 * GCOLS * 128), jnp.float32),
        pltpu.VMEM((2 * GCOLS * 16 * 128,), jnp.float32),
        pltpu.VMEM((2 * GCOLS * 16 * 128,), jnp.float32),
        pltpu.VMEM((GCOLS * 8 * 272,), jnp.float32),
        pltpu.VMEM((GCOLS * 8 * 272,), jnp.float32),
        pltpu.SemaphoreType.DMA,
        pltpu.SemaphoreType.DMA,
    ],
)


@functools.lru_cache(maxsize=None)
def _build_lookup(batch: int):
    n = batch * NUM_CAT
    per_w = n // NW
    nchunk = per_w // CHUNK
    assert per_w % CHUNK == 0

    def body(idx_hbm, mu_hbm, ps_hbm, mu_out, sig_out,
             idx_v, mu_v, ps_v, sem):
        wid = lax.axis_index("s") * 2 + lax.axis_index("c")
        base = wid * per_w

        def chunk_body(ci, carry):
            fb = base + ci * CHUNK
            pltpu.sync_copy(idx_hbm.at[pl.ds(fb, CHUNK)], idx_v)

            copies = []
            for j in range(NG):
                isl = idx_v.at[pl.ds(j * GSIZE, GSIZE)]
                dsl = pl.ds(j * GSIZE, GSIZE)
                copies.append(pltpu.async_copy(mu_hbm.at[isl], mu_v.at[dsl], sem))
                copies.append(pltpu.async_copy(ps_hbm.at[isl], ps_v.at[dsl], sem))
            for cp in copies:
                cp.wait()

            pltpu.sync_copy(mu_v, mu_out.at[pl.ds(fb, CHUNK)])

            def sp_body(r, c2):
                v = ps_v[r, :]
                e = jnp.exp(-jnp.abs(v))
                p = jnp.float32(_C[6])
                for coef in (_C[5], _C[4], _C[3], _C[2], _C[1], _C[0]):
                    p = p * e + jnp.float32(coef)
                ps_v[r, :] = jnp.maximum(v, 0.0) + p
                return c2

            lax.fori_loop(0, CHUNK, sp_body, 0, unroll=4)

            pltpu.sync_copy(ps_v, sig_out.at[pl.ds(fb, CHUNK)])
            return carry

        lax.fori_loop(0, nchunk, chunk_body, 0)

    return pl.kernel(
        body,
        out_type=(
            jax.ShapeDtypeStruct((n, OUT_DIMS), jnp.float32),
            jax.ShapeDtypeStruct((n, OUT_DIMS), jnp.float32),
        ),
        mesh=plsc.VectorSubcoreMesh(core_axis_name="c", subcore_axis_name="s"),
        compiler_params=pltpu.CompilerParams(use_tc_tiling_on_sc=False),
        scratch_types=[
            pltpu.VMEM((CHUNK,), jnp.int32),
            pltpu.VMEM((CHUNK, OUT_DIMS), jnp.float32),
            pltpu.VMEM((CHUNK, OUT_DIMS), jnp.float32),
            pltpu.SemaphoreType.DMA,
        ],
    )


def kernel(x, mask, mu_embeddings, presig_embeddings):
    batch = x.shape[0]
    shift = (jnp.arange(NUM_CAT, dtype=jnp.int32) * (MOST_CAT + 1))[None, :]
    idx = ((x + 1.0) * mask).astype(jnp.int32) + shift
    def tail(t):
        return jnp.pad(t[TCOLS * 128:], ((0, 128 - TEDGE), (0, 0))).reshape(-1)

    mu_lin, ps_lin = _detile(jnp.swapaxes(mu_embeddings, 0, 1),
                             jnp.swapaxes(presig_embeddings, 0, 1),
                             tail(mu_embeddings), tail(presig_embeddings))
    mu_flat, sig_flat = _build_lookup(batch)(
        idx.reshape(-1),
        mu_lin.reshape(ROWS_PAD, OUT_DIMS),
        ps_lin.reshape(ROWS_PAD, OUT_DIMS))
    return (mu_flat.reshape(batch, NUM_CAT * OUT_DIMS),
            sig_flat.reshape(batch, NUM_CAT * OUT_DIMS))


# speedup vs baseline: 1.4351x; 1.4351x over previous
import functools

import jax
import jax.numpy as jnp
from jax import lax
from jax.experimental import pallas as pl
from jax.experimental.pallas import tpu as pltpu
from jax.experimental.pallas import tpu_sc as plsc

NUM_CAT = 26
OUT_DIMS = 16
MOST_CAT = 100000
MIN_SIG = 1e-4

ROWS = NUM_CAT * (MOST_CAT + 1)
TCOLS = ROWS // 128
TEDGE = ROWS - TCOLS * 128
ROWS_PAD = (TCOLS + 1) * 128

NW = 32
CHUNK = 1024
GSIZE = 128
NG = CHUNK // GSIZE

GCOLS = 4
NGRP = TCOLS // GCOLS
GPW = -(-NGRP // NW)

_C = (
    3.5075520537e-06 + MIN_SIG,
    0.99979243573,
    -0.49697791117,
    0.31459053537,
    -0.18878267362,
    0.081726808375,
    -0.017208061121,
)


def _transpose_pair(tv_mu, tv_ps, ov_mu, ov_ps, tb, ob, ncols):
    idxs = [lax.iota(jnp.int32, 16) * OUT_DIMS + d for d in range(OUT_DIMS)]

    def blk_body(b, carry):
        c0 = tb + b * 16
        o0 = ob + b * 256
        ovb_mu = ov_mu.at[pl.ds(o0, 256)]
        ovb_ps = ov_ps.at[pl.ds(o0, 256)]
        for d in range(OUT_DIMS):
            vm = tv_mu[d, pl.ds(c0, 16)]
            vp = tv_ps[d, pl.ds(c0, 16)]
            plsc.store_scatter(ovb_mu, [idxs[d]], vm)
            plsc.store_scatter(ovb_ps, [idxs[d]], vp)
        return carry

    lax.fori_loop(0, ncols // 16, blk_body, 0, unroll=4)


GW = GCOLS * 128
GWORDS = GW * OUT_DIMS


def _detile_body(mu_t, ps_t, mu_tail, ps_tail, mu_lin, ps_lin,
                 tv_mu, tv_ps, ov_mu, ov_ps, sem_in, sem_out):
    wid = lax.axis_index("s") * 2 + lax.axis_index("c")
    g_lo = wid * GPW
    g_hi = jnp.minimum(g_lo + GPW, NGRP)

    def fire_in(g):
        par = (g - g_lo) % 2
        c0 = g * GW
        pltpu.async_copy(mu_t.at[:, pl.ds(c0, GW)],
                         tv_mu.at[:, pl.ds(par * GW, GW)], sem_in)
        pltpu.async_copy(ps_t.at[:, pl.ds(c0, GW)],
                         tv_ps.at[:, pl.ds(par * GW, GW)], sem_in)

    def drain_in(g):
        c0 = g * GW
        pltpu.make_async_copy(mu_t.at[:, pl.ds(c0, GW)],
                              tv_mu.at[:, pl.ds(0, GW)], sem_in).wait()
        pltpu.make_async_copy(ps_t.at[:, pl.ds(c0, GW)],
                              tv_ps.at[:, pl.ds(0, GW)], sem_in).wait()

    def fire_out(g):
        par = (g - g_lo) % 2
        w0 = g * GWORDS
        pltpu.async_copy(ov_mu.at[pl.ds(par * GWORDS, GWORDS)],
                         mu_lin.at[pl.ds(w0, GWORDS)], sem_out)
        pltpu.async_copy(ov_ps.at[pl.ds(par * GWORDS, GWORDS)],
                         ps_lin.at[pl.ds(w0, GWORDS)], sem_out)

    def drain_out(g):
        w0 = g * GWORDS
        pltpu.make_async_copy(ov_mu.at[pl.ds(0, GWORDS)],
                              mu_lin.at[pl.ds(w0, GWORDS)], sem_out).wait()
        pltpu.make_async_copy(ov_ps.at[pl.ds(0, GWORDS)],
                              ps_lin.at[pl.ds(w0, GWORDS)], sem_out).wait()

    fire_in(g_lo)

    def grp_body(g, carry):
        par = (g - g_lo) % 2

        @pl.when(g + 1 < g_hi)
        def _():
            fire_in(g + 1)

        drain_in(g)

        @pl.when(g - 2 >= g_lo)
        def _():
            drain_out(g - 2)

        _transpose_pair(tv_mu, tv_ps, ov_mu, ov_ps,
                        par * GW, par * GWORDS, GW)
        fire_out(g)
        return carry

    lax.fori_loop(g_lo, g_hi, grp_body, 0)
    drain_out(g_hi - 2)
    drain_out(g_hi - 1)

    @pl.when(wid == NW - 1)
    def _edge():
        w0 = TCOLS * 16 * 128
        pltpu.sync_copy(mu_tail, ov_mu.at[pl.ds(0, 2048)])
        pltpu.sync_copy(ov_mu.at[pl.ds(0, 2048)], mu_lin.at[pl.ds(w0, 2048)])
        pltpu.sync_copy(ps_tail, ov_ps.at[pl.ds(0, 2048)])
        pltpu.sync_copy(ov_ps.at[pl.ds(0, 2048)], ps_lin.at[pl.ds(w0, 2048)])


_detile = pl.kernel(
    _detile_body,
    out_type=(
        jax.ShapeDtypeStruct((ROWS_PAD * OUT_DIMS,), jnp.float32),
        jax.ShapeDtypeStruct((ROWS_PAD * OUT_DIMS,), jnp.float32),
    ),
    mesh=plsc.VectorSubcoreMesh(core_axis_name="c", subcore_axis_name="s"),
    compiler_params=pltpu.CompilerParams(use_tc_tiling_on_sc=True,
                                         needs_layout_passes=False),
    scratch_types=[
        pltpu.VMEM((16, 2 * GCOLS * 128), jnp.float32),
        pltpu.VMEM((16, 2 * GCOLS * 128), jnp.float32),
        pltpu.VMEM((2 * GCOLS * 16 * 128,), jnp.float32),
        pltpu.VMEM((2 * GCOLS * 16 * 128,), jnp.float32),
        pltpu.SemaphoreType.DMA,
        pltpu.SemaphoreType.DMA,
    ],
)


@functools.lru_cache(maxsize=None)
def _build_lookup(batch: int):
    n = batch * NUM_CAT
    per_w = n // NW
    nchunk = per_w // CHUNK
    assert per_w % CHUNK == 0

    def body(idx_hbm, mu_hbm, ps_hbm, mu_out, sig_out,
             idx_v, mu_v, ps_v, sem_i, sem_g, sem_o):
        wid = lax.axis_index("s") * 2 + lax.axis_index("c")
        base = wid * per_w

        def fire_chunk(ci):
            par = (ci % 2) * CHUNK
            fb = base + ci * CHUNK
            pltpu.async_copy(idx_hbm.at[pl.ds(fb, CHUNK)],
                             idx_v.at[pl.ds(par, CHUNK)], sem_i).wait()
            for j in range(NG):
                isl = idx_v.at[pl.ds(par + j * GSIZE, GSIZE)]
                dsl = pl.ds(par + j * GSIZE, GSIZE)
                pltpu.async_copy(mu_hbm.at[isl], mu_v.at[dsl], sem_g)
                pltpu.async_copy(ps_hbm.at[isl], ps_v.at[dsl], sem_g)

        def drain_gathers():
            for _ in range(2 * NG):
                pltpu.make_async_copy(mu_hbm.at[idx_v.at[pl.ds(0, GSIZE)]],
                                      mu_v.at[pl.ds(0, GSIZE)], sem_g).wait()

        def drain_out(ci):
            fb = base + ci * CHUNK
            pltpu.make_async_copy(mu_v.at[pl.ds(0, CHUNK)],
                                  mu_out.at[pl.ds(fb, CHUNK)], sem_o).wait()
            pltpu.make_async_copy(ps_v.at[pl.ds(0, CHUNK)],
                                  sig_out.at[pl.ds(fb, CHUNK)], sem_o).wait()

        fire_chunk(0)

        def chunk_body(ci, carry):
            par = (ci % 2) * CHUNK
            fb = base + ci * CHUNK
            drain_gathers()

            @pl.when(ci - 1 >= 0)
            def _():
                drain_out(ci - 1)

            @pl.when(ci + 1 < nchunk)
            def _():
                fire_chunk(ci + 1)

            pltpu.async_copy(mu_v.at[pl.ds(par, CHUNK)],
                             mu_out.at[pl.ds(fb, CHUNK)], sem_o)

            def sp_body(r, c2):
                v = ps_v[par + r, :]
                e = jnp.exp(-jnp.abs(v))
                p = jnp.float32(_C[6])
                for coef in (_C[5], _C[4], _C[3], _C[2], _C[1], _C[0]):
                    p = p * e + jnp.float32(coef)
                ps_v[par + r, :] = jnp.maximum(v, 0.0) + p
                return c2

            lax.fori_loop(0, CHUNK, sp_body, 0, unroll=4)

            pltpu.async_copy(ps_v.at[pl.ds(par, CHUNK)],
                             sig_out.at[pl.ds(fb, CHUNK)], sem_o)
            return carry

        lax.fori_loop(0, nchunk, chunk_body, 0)
        drain_out(nchunk - 1)

    return pl.kernel(
        body,
        out_type=(
            jax.ShapeDtypeStruct((n, OUT_DIMS), jnp.float32),
            jax.ShapeDtypeStruct((n, OUT_DIMS), jnp.float32),
        ),
        mesh=plsc.VectorSubcoreMesh(core_axis_name="c", subcore_axis_name="s"),
        compiler_params=pltpu.CompilerParams(use_tc_tiling_on_sc=False),
        scratch_types=[
            pltpu.VMEM((2 * CHUNK,), jnp.int32),
            pltpu.VMEM((2 * CHUNK, OUT_DIMS), jnp.float32),
            pltpu.VMEM((2 * CHUNK, OUT_DIMS), jnp.float32),
            pltpu.SemaphoreType.DMA,
            pltpu.SemaphoreType.DMA,
            pltpu.SemaphoreType.DMA,
        ],
    )


def kernel(x, mask, mu_embeddings, presig_embeddings):
    batch = x.shape[0]
    shift = (jnp.arange(NUM_CAT, dtype=jnp.int32) * (MOST_CAT + 1))[None, :]
    idx = ((x + 1.0) * mask).astype(jnp.int32) + shift
    def tail(t):
        return jnp.pad(t[TCOLS * 128:], ((0, 128 - TEDGE), (0, 0))).reshape(-1)

    mu_lin, ps_lin = _detile(jnp.swapaxes(mu_embeddings, 0, 1),
                             jnp.swapaxes(presig_embeddings, 0, 1),
                             tail(mu_embeddings), tail(presig_embeddings))
    mu_flat, sig_flat = _build_lookup(batch)(
        idx.reshape(-1),
        mu_lin.reshape(ROWS_PAD, OUT_DIMS),
        ps_lin.reshape(ROWS_PAD, OUT_DIMS))
    return (mu_flat.reshape(batch, NUM_CAT * OUT_DIMS),
            sig_flat.reshape(batch, NUM_CAT * OUT_DIMS))
